# baseline (speedup 1.0000x reference)
import jax
import jax.numpy as jnp
from jax import lax
from jax.experimental import pallas as pl
from jax.experimental.pallas import tpu as pltpu
from jax.experimental.pallas import tpu_sc as plsc

N = 10000
E = 320000
D = 128

NC = 2
NS = 16
L = 16

H = N // NC
H_PAD = 5008
ACC_ROWS = H_PAD + NS
EPT = E // NS
SUB = 4000
NSUB = EPT // SUB
G = 32
NGRP = SUB // G
RING = 8
IRING = 2 * RING

RPT = 312
LAST_ROWS = H - 15 * RPT
ZPT = 312
LAST_ZROWS = ACC_ROWS - 15 * ZPT


def _scale_store_rows(rows_v, wv, n_rows):
  def body(i, _):
    for j in range(D // L):
      sl = pl.ds(j * L, L)
      rows_v[0, i, sl] = rows_v[0, i, sl] * wv[sl]
    return 0

  lax.fori_loop(0, n_rows, body, 0)


def _sc_kernel(ge_hbm, src_hbm, dst_hbm, ef_hbm, w_hbm,
               out1_hbm, out0_hbm,
               src_v, dst_v, ef_v, rows_v, idx1_v, idx0_v, wv,
               gsem, ssem,
               acc1, acc0):
  c = lax.axis_index("c")
  s = lax.axis_index("s")
  lo = c * H
  dump = H_PAD + s
  ebase = s * EPT

  pltpu.sync_copy(w_hbm, wv)

  zero = jnp.zeros((L,), jnp.float32)

  def zrow(i, _):
    for j in range(D // L):
      rows_v[0, i, pl.ds(j * L, L)] = zero
    return 0

  lax.fori_loop(0, G, zrow, 0)

  def zero_acc_slice(n_rows):
    zbase = s * ZPT
    for acc in (acc1, acc0):
      off = 0
      while off < n_rows:
        ck = min(G, n_rows - off)
        pltpu.sync_copy(rows_v.at[0, pl.ds(0, ck)],
                        acc.at[pl.ds(zbase + off, ck)])
        off += ck

  @pl.when(s < NS - 1)
  def _():
    zero_acc_slice(ZPT)

  @pl.when(s == NS - 1)
  def _():
    zero_acc_slice(LAST_ZROWS)

  plsc.subcore_barrier()

  dumpvec = jnp.zeros((L,), jnp.int32) + dump

  def compute_idx(g):
    eb = g * G
    b = g % IRING
    for j in range(G // L):
      sl = pl.ds(eb + j * L, L)
      dv = dst_v[sl]
      ev = ef_v[sl]
      inr = (dv >= lo) & (dv < lo + H)
      dl = dv - lo
      idx1_v[b, pl.ds(j * L, L)] = jnp.where(inr, dl, dumpvec)
      idx0_v[b, pl.ds(j * L, L)] = jnp.where(inr & (ev == 0), dl, dumpvec)

  def fire_gather(g):
    b = g % RING
    pltpu.async_copy(
        ge_hbm.at[src_v.at[pl.ds(g * G, G)]], rows_v.at[b], gsem.at[b])

  def wait_gather(g):
    b = g % RING
    pltpu.make_async_copy(
        ge_hbm.at[src_v.at[pl.ds(g * G, G)]], rows_v.at[b], gsem.at[b]).wait()

  def fire_scatters(g):
    b = g % RING
    ib = g % IRING
    s1 = pltpu.async_copy(rows_v.at[b], acc1.at[idx1_v.at[ib]], ssem.at[b],
                          add=True)
    s0 = pltpu.async_copy(rows_v.at[b], acc0.at[idx0_v.at[ib]], ssem.at[b],
                          add=True)
    return s1, s0

  def sub(sc, _):
    soff = ebase + sc * SUB
    pltpu.sync_copy(src_hbm.at[pl.ds(soff, SUB)], src_v)
    pltpu.sync_copy(dst_hbm.at[pl.ds(soff, SUB)], dst_v)
    pltpu.sync_copy(ef_hbm.at[pl.ds(soff, SUB)], ef_v)

    dyn0 = jnp.minimum(dst_v[pl.ds(0, L)][0] * 0, 0)

    def vcopy(i, _):
      src_v[pl.ds(i * L, L)] = src_v[pl.ds(i * L, L)] + dyn0
      return 0

    lax.fori_loop(0, SUB // L, vcopy, 0)

    for g in range(RING):
      compute_idx(g)
      fire_gather(g)

    def step(g, _):
      wait_gather(g)
      s1, s0 = fire_scatters(g)
      compute_idx(g + RING)
      s1.wait()
      s0.wait()
      fire_gather(g + RING)
      return 0

    lax.fori_loop(0, NGRP - RING + dyn0, step, 0)

    for k in range(RING):
      g = NGRP - RING + k
      wait_gather(g)
      s1, s0 = fire_scatters(g)
      s1.wait()
      s0.wait()
    return 0

  lax.fori_loop(0, NSUB, sub, 0)
  plsc.subcore_barrier()

  def flush(acc, out_hbm, n_rows):
    obase = lo + s * RPT
    off = 0
    while off < n_rows:
      ck = min(G, n_rows - off)
      pltpu.sync_copy(acc.at[pl.ds(s * RPT + off, ck)],
                      rows_v.at[0, pl.ds(0, ck)])
      _scale_store_rows(rows_v, wv, ck)
      pltpu.sync_copy(rows_v.at[0, pl.ds(0, ck)],
                      out_hbm.at[pl.ds(obase + off, ck)])
      off += ck

  @pl.when(s < NS - 1)
  def _():
    flush(acc1, out1_hbm, RPT)
    flush(acc0, out0_hbm, RPT)

  @pl.when(s == NS - 1)
  def _():
    flush(acc1, out1_hbm, LAST_ROWS)
    flush(acc0, out0_hbm, LAST_ROWS)


@jax.jit
def _run(graph_embedding, src, dst, e_feat, w):
  mesh = plsc.VectorSubcoreMesh(core_axis_name="c", subcore_axis_name="s")
  f = pl.kernel(
      _sc_kernel,
      out_type=(
          jax.ShapeDtypeStruct((N, D), jnp.float32),
          jax.ShapeDtypeStruct((N, D), jnp.float32),
      ),
      mesh=mesh,
      scratch_types=[
          pltpu.VMEM((SUB,), jnp.int32),
          pltpu.VMEM((SUB,), jnp.int32),
          pltpu.VMEM((SUB,), jnp.int32),
          pltpu.VMEM((RING, G, D), jnp.float32),
          pltpu.VMEM((IRING, G), jnp.int32),
          pltpu.VMEM((IRING, G), jnp.int32),
          pltpu.VMEM((D,), jnp.float32),
          pltpu.SemaphoreType.DMA((RING,)),
          pltpu.SemaphoreType.DMA((RING,)),
          pltpu.VMEM_SHARED((ACC_ROWS, D), jnp.float32),
          pltpu.VMEM_SHARED((ACC_ROWS, D), jnp.float32),
      ],
  )
  return f(graph_embedding, src, dst, e_feat, w)


def kernel(graph_embedding, edge_index, e_feat, weight):
  src = edge_index[0]
  dst = edge_index[1]
  w = weight.reshape(D)
  res, res0 = _run(graph_embedding, src, dst, e_feat, w)
  return (res, res0, res0, res0, res0)


# speedup vs baseline: 1.2125x; 1.2125x over previous
import jax
import jax.numpy as jnp
from jax import lax
from jax.experimental import pallas as pl
from jax.experimental.pallas import tpu as pltpu
from jax.experimental.pallas import tpu_sc as plsc

N = 10000
E = 320000
D = 128

NC = 2
NS = 16
L = 16

H = N // NC
H_PAD = 5008
ACC_ROWS = H_PAD + NS
EPT = E // NS
SUB = 4000
NSUB = EPT // SUB
G = 32
RING = 5
IRING = 2 * RING
LISTSZ = SUB + G * (RING + 1)

RPT = 312
LAST_ROWS = H - 15 * RPT
ZPT = 312
LAST_ZROWS = ACC_ROWS - 15 * ZPT


def _scale_store_rows(rows_v, wv, n_rows):
  def body(i, _):
    for j in range(D // L):
      sl = pl.ds(j * L, L)
      rows_v[0, i, sl] = rows_v[0, i, sl] * wv[sl]
    return 0

  lax.fori_loop(0, n_rows, body, 0)


def _sc_kernel(ge_hbm, src_hbm, dst_hbm, ef_hbm, w_hbm,
               out1_hbm, out0_hbm,
               src_v, dst_v, ef_v, srcC_v, dlC_v, d0C_v,
               rows_v, idx1_v, idx0_v, wv,
               gsem, ssem,
               acc1, acc0):
  c = lax.axis_index("c")
  s = lax.axis_index("s")
  lo = c * H
  dump = H_PAD + s
  ebase = s * EPT

  pltpu.sync_copy(w_hbm, wv)

  zero = jnp.zeros((L,), jnp.float32)

  def zrow(i, _):
    for j in range(D // L):
      rows_v[0, i, pl.ds(j * L, L)] = zero
    return 0

  lax.fori_loop(0, G, zrow, 0)

  def zero_acc_slice(n_rows):
    zbase = s * ZPT
    for acc in (acc1, acc0):
      off = 0
      while off < n_rows:
        ck = min(G, n_rows - off)
        pltpu.sync_copy(rows_v.at[0, pl.ds(0, ck)],
                        acc.at[pl.ds(zbase + off, ck)])
        off += ck

  @pl.when(s < NS - 1)
  def _():
    zero_acc_slice(ZPT)

  @pl.when(s == NS - 1)
  def _():
    zero_acc_slice(LAST_ZROWS)

  plsc.subcore_barrier()

  dumpvec = jnp.zeros((L,), jnp.int32) + dump
  zsrc = jnp.zeros((L,), jnp.int32)
  tmask = zsrc == 0
  iota = lax.iota(jnp.int32, L)

  def copy_idx(g):
    b = g % IRING
    eb = g * G
    for j in range(G // L):
      idx1_v[b, pl.ds(j * L, L)] = dlC_v[pl.ds(eb + j * L, L)]
      idx0_v[b, pl.ds(j * L, L)] = d0C_v[pl.ds(eb + j * L, L)]

  def fire_gather(g):
    b = g % RING
    pltpu.async_copy(
        ge_hbm.at[srcC_v.at[pl.ds(g * G, G)]], rows_v.at[b], gsem.at[b])

  def wait_gather(g):
    b = g % RING
    pltpu.make_async_copy(
        ge_hbm.at[srcC_v.at[pl.ds(g * G, G)]], rows_v.at[b],
        gsem.at[b]).wait()

  def fire_scatters(g):
    b = g % RING
    ib = g % IRING
    s1 = pltpu.async_copy(rows_v.at[b], acc1.at[idx1_v.at[ib]], ssem.at[b],
                          add=True)
    s0 = pltpu.async_copy(rows_v.at[b], acc0.at[idx0_v.at[ib]], ssem.at[b],
                          add=True)
    return s1, s0

  def sub(sc, _):
    soff = ebase + sc * SUB
    pltpu.sync_copy(src_hbm.at[pl.ds(soff, SUB)], src_v)
    pltpu.sync_copy(dst_hbm.at[pl.ds(soff, SUB)], dst_v)
    pltpu.sync_copy(ef_hbm.at[pl.ds(soff, SUB)], ef_v)

    def scan_step(i, ptr):
      sl = pl.ds(i * L, L)
      dv = dst_v[sl]
      sv = src_v[sl]
      ev = ef_v[sl]
      m = (dv >= lo) & (dv < lo + H)
      mi = m.astype(jnp.int32)
      pos = plsc.cumsum(mi) - mi + ptr
      dl = dv - lo
      plsc.store_scatter(srcC_v, [pos], sv, mask=m)
      plsc.store_scatter(dlC_v, [pos], dl, mask=m)
      plsc.store_scatter(d0C_v, [pos], jnp.where(ev == 0, dl, dumpvec),
                         mask=m)
      return ptr + jnp.sum(mi)

    cnt = lax.fori_loop(0, SUB // L, scan_step, jnp.int32(0))

    for k in range(G * (RING + 1) // L):
      at = [iota + (cnt + k * L)]
      plsc.store_scatter(srcC_v, at, zsrc, mask=tmask)
      plsc.store_scatter(dlC_v, at, dumpvec, mask=tmask)
      plsc.store_scatter(d0C_v, at, dumpvec, mask=tmask)

    ng = jnp.maximum((cnt + G - 1) // G, RING)

    for g in range(RING):
      copy_idx(g)
      fire_gather(g)

    def step(g, _):
      wait_gather(g)
      s1, s0 = fire_scatters(g)
      copy_idx(g + RING)
      s1.wait()
      s0.wait()
      fire_gather(g + RING)
      return 0

    lax.fori_loop(0, ng - RING, step, 0)

    for k in range(RING):
      g = ng - RING + k
      wait_gather(g)
      s1, s0 = fire_scatters(g)
      s1.wait()
      s0.wait()
    return 0

  lax.fori_loop(0, NSUB, sub, 0)
  plsc.subcore_barrier()

  def flush(acc, out_hbm, n_rows):
    obase = lo + s * RPT
    off = 0
    while off < n_rows:
      ck = min(G, n_rows - off)
      pltpu.sync_copy(acc.at[pl.ds(s * RPT + off, ck)],
                      rows_v.at[0, pl.ds(0, ck)])
      _scale_store_rows(rows_v, wv, ck)
      pltpu.sync_copy(rows_v.at[0, pl.ds(0, ck)],
                      out_hbm.at[pl.ds(obase + off, ck)])
      off += ck

  @pl.when(s < NS - 1)
  def _():
    flush(acc1, out1_hbm, RPT)
    flush(acc0, out0_hbm, RPT)

  @pl.when(s == NS - 1)
  def _():
    flush(acc1, out1_hbm, LAST_ROWS)
    flush(acc0, out0_hbm, LAST_ROWS)


@jax.jit
def _run(graph_embedding, src, dst, e_feat, w):
  mesh = plsc.VectorSubcoreMesh(core_axis_name="c", subcore_axis_name="s")
  f = pl.kernel(
      _sc_kernel,
      out_type=(
          jax.ShapeDtypeStruct((N, D), jnp.float32),
          jax.ShapeDtypeStruct((N, D), jnp.float32),
      ),
      mesh=mesh,
      compiler_params=pltpu.CompilerParams(needs_layout_passes=False),
      scratch_types=[
          pltpu.VMEM((SUB,), jnp.int32),
          pltpu.VMEM((SUB,), jnp.int32),
          pltpu.VMEM((SUB,), jnp.int32),
          pltpu.VMEM((LISTSZ,), jnp.int32),
          pltpu.VMEM((LISTSZ,), jnp.int32),
          pltpu.VMEM((LISTSZ,), jnp.int32),
          pltpu.VMEM((RING, G, D), jnp.float32),
          pltpu.VMEM((IRING, G), jnp.int32),
          pltpu.VMEM((IRING, G), jnp.int32),
          pltpu.VMEM((D,), jnp.float32),
          pltpu.SemaphoreType.DMA((RING,)),
          pltpu.SemaphoreType.DMA((RING,)),
          pltpu.VMEM_SHARED((ACC_ROWS, D), jnp.float32),
          pltpu.VMEM_SHARED((ACC_ROWS, D), jnp.float32),
      ],
  )
  return f(graph_embedding, src, dst, e_feat, w)


def kernel(graph_embedding, edge_index, e_feat, weight):
  src = edge_index[0]
  dst = edge_index[1]
  w = weight.reshape(D)
  res, res0 = _run(graph_embedding, src, dst, e_feat, w)
  return (res, res0, res0, res0, res0)


# speedup vs baseline: 1.2611x; 1.0401x over previous
import jax
import jax.numpy as jnp
from jax import lax
from jax.experimental import pallas as pl
from jax.experimental.pallas import tpu as pltpu
from jax.experimental.pallas import tpu_sc as plsc

N = 10000
E = 320000
D = 128

NC = 2
NS = 16
L = 16

H = N // NC
H_PAD = 5008
ACC_ROWS = H_PAD + NS
EPT = E // NS
SUB = 4000
NSUB = EPT // SUB
G = 32
RING = 5
IRING = 2 * RING
LISTSZ = SUB + G * (RING + 1)

RPT = 312
LAST_ROWS = H - 15 * RPT
ZPT = 312
LAST_ZROWS = ACC_ROWS - 15 * ZPT


def _scale_store_rows(rows_v, wv, n_rows):
  def body(i, _):
    for j in range(D // L):
      sl = pl.ds(j * L, L)
      rows_v[0, i, sl] = rows_v[0, i, sl] * wv[sl]
    return 0

  lax.fori_loop(0, n_rows, body, 0)


def _sc_kernel(ge_hbm, src_hbm, dst_hbm, ef_hbm, w_hbm,
               out1_hbm, out0_hbm,
               src_v, dst_v, ef_v, srcC_v, dlC_v, d0C_v,
               rows_v, idx1_v, idx0_v, wv,
               gsem, ssem, tsem,
               acc1, acc0):
  c = lax.axis_index("c")
  s = lax.axis_index("s")
  lo = c * H
  dump = H_PAD + s
  ebase = s * EPT

  pltpu.sync_copy(w_hbm, wv)

  zero = jnp.zeros((L,), jnp.float32)

  def zrow(i, _):
    for j in range(D // L):
      rows_v[0, i, pl.ds(j * L, L)] = zero
    return 0

  lax.fori_loop(0, G, zrow, 0)

  def zero_acc_slice(n_rows):
    zbase = s * ZPT
    for acc in (acc1, acc0):
      off = 0
      while off < n_rows:
        ck = min(G, n_rows - off)
        pltpu.sync_copy(rows_v.at[0, pl.ds(0, ck)],
                        acc.at[pl.ds(zbase + off, ck)])
        off += ck

  @pl.when(s < NS - 1)
  def _():
    zero_acc_slice(ZPT)

  @pl.when(s == NS - 1)
  def _():
    zero_acc_slice(LAST_ZROWS)

  plsc.subcore_barrier()

  dumpvec = jnp.zeros((L,), jnp.int32) + dump
  zsrc = jnp.zeros((L,), jnp.int32)
  tmask = zsrc == 0
  iota = lax.iota(jnp.int32, L)

  def copy_idx(g):
    b = g % IRING
    eb = g * G
    for j in range(G // L):
      idx1_v[b, pl.ds(j * L, L)] = dlC_v[pl.ds(eb + j * L, L)]
      idx0_v[b, pl.ds(j * L, L)] = d0C_v[pl.ds(eb + j * L, L)]

  def fire_gather(g):
    b = g % RING
    pltpu.async_copy(
        ge_hbm.at[srcC_v.at[pl.ds(g * G, G)]], rows_v.at[b], gsem.at[b])

  def wait_gather(g):
    b = g % RING
    pltpu.make_async_copy(
        ge_hbm.at[srcC_v.at[pl.ds(g * G, G)]], rows_v.at[b],
        gsem.at[b]).wait()

  def fire_scatters(g):
    b = g % RING
    ib = g % IRING
    s1 = pltpu.async_copy(rows_v.at[b], acc1.at[idx1_v.at[ib]], ssem.at[b],
                          add=True)
    s0 = pltpu.async_copy(rows_v.at[b], acc0.at[idx0_v.at[ib]], ssem.at[b],
                          add=True)
    return s1, s0

  def fire_stage(sc):
    soff = ebase + sc * SUB
    pltpu.async_copy(src_hbm.at[pl.ds(soff, SUB)], src_v, tsem)
    pltpu.async_copy(dst_hbm.at[pl.ds(soff, SUB)], dst_v, tsem)
    pltpu.async_copy(ef_hbm.at[pl.ds(soff, SUB)], ef_v, tsem)

  def wait_stage(sc):
    soff = ebase + sc * SUB
    pltpu.make_async_copy(src_hbm.at[pl.ds(soff, SUB)], src_v, tsem).wait()
    pltpu.make_async_copy(dst_hbm.at[pl.ds(soff, SUB)], dst_v, tsem).wait()
    pltpu.make_async_copy(ef_hbm.at[pl.ds(soff, SUB)], ef_v, tsem).wait()

  fire_stage(0)
  wait_stage(0)

  def sub(sc, _):
    def scan_step(i, ptr):
      sl = pl.ds(i * L, L)
      dv = dst_v[sl]
      sv = src_v[sl]
      ev = ef_v[sl]
      m = (dv >= lo) & (dv < lo + H)
      mi = m.astype(jnp.int32)
      pos = plsc.cumsum(mi) - mi + ptr
      dl = dv - lo
      plsc.store_scatter(srcC_v, [pos], sv, mask=m)
      plsc.store_scatter(dlC_v, [pos], dl, mask=m)
      plsc.store_scatter(d0C_v, [pos], jnp.where(ev == 0, dl, dumpvec),
                         mask=m)
      return ptr + jnp.sum(mi)

    cnt = lax.fori_loop(0, SUB // L, scan_step, jnp.int32(0))

    for k in range(G * (RING + 1) // L):
      at = [iota + (cnt + k * L)]
      plsc.store_scatter(srcC_v, at, zsrc, mask=tmask)
      plsc.store_scatter(dlC_v, at, dumpvec, mask=tmask)
      plsc.store_scatter(d0C_v, at, dumpvec, mask=tmask)

    ng = jnp.maximum((cnt + G - 1) // G, RING)

    @pl.when(sc < NSUB - 1)
    def _():
      fire_stage(sc + 1)

    for g in range(RING):
      copy_idx(g)
      fire_gather(g)

    def step(g, _):
      wait_gather(g)
      s1, s0 = fire_scatters(g)
      copy_idx(g + RING)
      s1.wait()
      s0.wait()
      fire_gather(g + RING)
      return 0

    lax.fori_loop(0, ng - RING, step, 0)

    for k in range(RING):
      g = ng - RING + k
      wait_gather(g)
      s1, s0 = fire_scatters(g)
      s1.wait()
      s0.wait()

    @pl.when(sc < NSUB - 1)
    def _():
      wait_stage(sc + 1)
    return 0

  lax.fori_loop(0, NSUB, sub, 0)
  plsc.subcore_barrier()

  def flush(acc, out_hbm, n_rows):
    obase = lo + s * RPT
    off = 0
    while off < n_rows:
      ck = min(G, n_rows - off)
      pltpu.sync_copy(acc.at[pl.ds(s * RPT + off, ck)],
                      rows_v.at[0, pl.ds(0, ck)])
      _scale_store_rows(rows_v, wv, ck)
      pltpu.sync_copy(rows_v.at[0, pl.ds(0, ck)],
                      out_hbm.at[pl.ds(obase + off, ck)])
      off += ck

  @pl.when(s < NS - 1)
  def _():
    flush(acc1, out1_hbm, RPT)
    flush(acc0, out0_hbm, RPT)

  @pl.when(s == NS - 1)
  def _():
    flush(acc1, out1_hbm, LAST_ROWS)
    flush(acc0, out0_hbm, LAST_ROWS)


@jax.jit
def _run(graph_embedding, src, dst, e_feat, w):
  mesh = plsc.VectorSubcoreMesh(core_axis_name="c", subcore_axis_name="s")
  f = pl.kernel(
      _sc_kernel,
      out_type=(
          jax.ShapeDtypeStruct((N, D), jnp.float32),
          jax.ShapeDtypeStruct((N, D), jnp.float32),
      ),
      mesh=mesh,
      compiler_params=pltpu.CompilerParams(needs_layout_passes=False),
      scratch_types=[
          pltpu.VMEM((SUB,), jnp.int32),
          pltpu.VMEM((SUB,), jnp.int32),
          pltpu.VMEM((SUB,), jnp.int32),
          pltpu.VMEM((LISTSZ,), jnp.int32),
          pltpu.VMEM((LISTSZ,), jnp.int32),
          pltpu.VMEM((LISTSZ,), jnp.int32),
          pltpu.VMEM((RING, G, D), jnp.float32),
          pltpu.VMEM((IRING, G), jnp.int32),
          pltpu.VMEM((IRING, G), jnp.int32),
          pltpu.VMEM((D,), jnp.float32),
          pltpu.SemaphoreType.DMA((RING,)),
          pltpu.SemaphoreType.DMA((RING,)),
          pltpu.SemaphoreType.DMA,
          pltpu.VMEM_SHARED((ACC_ROWS, D), jnp.float32),
          pltpu.VMEM_SHARED((ACC_ROWS, D), jnp.float32),
      ],
  )
  return f(graph_embedding, src, dst, e_feat, w)


def kernel(graph_embedding, edge_index, e_feat, weight):
  src = edge_index[0]
  dst = edge_index[1]
  w = weight.reshape(D)
  res, res0 = _run(graph_embedding, src, dst, e_feat, w)
  return (res, res0, res0, res0, res0)
